# baseline (speedup 1.0000x reference)
import functools

import numpy as _np

import jax
import jax.numpy as jnp
from jax import lax
from jax.experimental import pallas as pl
from jax.experimental.pallas import tpu as pltpu
from jax.experimental.pallas import tpu_sc as plsc


def _silu(x):
    return x * jax.nn.sigmoid(x)


def _filter_body(rbf_ref, w1_ref, b1_ref, w2_ref, b2_ref, fa_ref, fb_ref):
    h = _silu(jnp.dot(rbf_ref[...], w1_ref[...],
                      preferred_element_type=jnp.float32) + b1_ref[...])
    f = jnp.dot(h, w2_ref[...], preferred_element_type=jnp.float32) + b2_ref[...]
    fb16 = f.astype(jnp.bfloat16)
    fa_ref[0] = fb16[:, 0:64]
    fa_ref[1] = fb16[:, 64:128]
    fb_ref[0] = fb16[:, 128:256]
    fb_ref[1] = fb16[:, 256:384]


def _filter_mlp(rbf, W_f1, b_f1, W_f2p, b_f2p, bE=2048):
    E, R = rbf.shape
    grid = (E // bE,)
    return pl.pallas_call(
        _filter_body,
        grid=grid,
        in_specs=[
            pl.BlockSpec((bE, R), lambda i: (i, 0)),
            pl.BlockSpec(W_f1.shape, lambda i: (0, 0)),
            pl.BlockSpec(b_f1.shape, lambda i: (0,)),
            pl.BlockSpec(W_f2p.shape, lambda i: (0, 0)),
            pl.BlockSpec(b_f2p.shape, lambda i: (0,)),
        ],
        out_specs=[
            pl.BlockSpec((2, bE, 64), lambda i: (0, i, 0)),
            pl.BlockSpec((2, bE, 128), lambda i: (0, i, 0)),
        ],
        out_shape=[
            jax.ShapeDtypeStruct((2, E, 64), jnp.bfloat16),
            jax.ShapeDtypeStruct((2, E, 128), jnp.bfloat16),
        ],
    )(rbf, W_f1, b_f1, W_f2p, b_f2p)


def _tables_body(s_ref, v_ref, w1_ref, b1_ref, w2_ref, b2_ref, t1_ref, t2_ref):
    bN = s_ref.shape[0]
    h = _silu(jnp.dot(s_ref[...], w1_ref[...],
                      preferred_element_type=jnp.float32) + b1_ref[...])
    t = jnp.dot(h, w2_ref[...], preferred_element_type=jnp.float32) + b2_ref[...]
    v = v_ref[...]
    pad = jnp.zeros((bN, 64), jnp.bfloat16)
    for c in range(2):
        tds = t[:, 64 * c:64 * c + 64]
        tvv = t[:, 128 + 64 * c:128 + 64 * c + 64]
        tvr = t[:, 256 + 64 * c:256 + 64 * c + 64]
        g0 = tvv * v[:, 0, 64 * c:64 * c + 64]
        g1 = tvv * v[:, 1, 64 * c:64 * c + 64]
        g2 = tvv * v[:, 2, 64 * c:64 * c + 64]
        t1_ref[c] = jnp.concatenate(
            [tds.astype(jnp.bfloat16), g0.astype(jnp.bfloat16),
             tvr.astype(jnp.bfloat16), pad], axis=-1)
        t2_ref[c] = jnp.concatenate(
            [g1.astype(jnp.bfloat16), g2.astype(jnp.bfloat16),
             tvr.astype(jnp.bfloat16), pad], axis=-1)


def _node_tables(s, v, W_s1, b_s1, W_s2, b_s2, bN=400):
    N, H = s.shape
    grid = (N // bN,)
    return pl.pallas_call(
        _tables_body,
        grid=grid,
        in_specs=[
            pl.BlockSpec((bN, H), lambda i: (i, 0)),
            pl.BlockSpec((bN, 3, H), lambda i: (i, 0, 0)),
            pl.BlockSpec(W_s1.shape, lambda i: (0, 0)),
            pl.BlockSpec(b_s1.shape, lambda i: (0,)),
            pl.BlockSpec(W_s2.shape, lambda i: (0, 0)),
            pl.BlockSpec(b_s2.shape, lambda i: (0,)),
        ],
        out_specs=[
            pl.BlockSpec((2, bN, 256), lambda i: (0, i, 0)),
            pl.BlockSpec((2, bN, 256), lambda i: (0, i, 0)),
        ],
        out_shape=[
            jax.ShapeDtypeStruct((2, N, 256), jnp.bfloat16),
            jax.ShapeDtypeStruct((2, N, 256), jnp.bfloat16),
        ],
    )(s, v, W_s1, b_s1, W_s2, b_s2)


_K = 64


def _sc_body(N, Np, E, fa_hbm, fb_hbm, t1_hbm, t2_hbm, src_hbm, dst_hbm,
             u0_hbm, u12_hbm, zeros_hbm, out1_hbm, out2_hbm,
             acc,
             sbuf0, sbuf1, dbuf0, dbuf1, gidx0, gidx1, sidx0, sidx1,
             ubuf0, ubuf1, fabuf0, fabuf1, fbbuf0, fbbuf1,
             gbuf0, gbuf1, obuf0, obuf1,
             semi0, semi1, semg0, semg1, sems0, sems1):
    c = lax.axis_index("c")
    sid = lax.axis_index("s")
    K = _K
    ept = E // 16
    nwin = ept // K
    tile_lo = sid * ept
    coff_e = c * E

    rows = Np // 16
    row_lo = sid * rows
    cNp = c * Np

    cN_vec = jnp.full((16,), c * N, jnp.int32)

    sbuf = [sbuf0, sbuf1]
    dbuf = [dbuf0, dbuf1]
    gidx = [gidx0, gidx1]
    sidx = [sidx0, sidx1]
    ubuf = [ubuf0, ubuf1]
    fabuf = [fabuf0, fabuf1]
    fbbuf = [fbbuf0, fbbuf1]
    gbuf = [gbuf0, gbuf1]
    obuf = [obuf0, obuf1]
    semi = [semi0, semi1]
    semg = [semg0, semg1]
    sems = [sems0, sems1]

    def zero_acc():
        pltpu.sync_copy(zeros_hbm.at[pl.ds(row_lo, rows)],
                        acc.at[pl.ds(row_lo, rows)])
        plsc.subcore_barrier()

    def dump_acc(out_hbm):
        plsc.subcore_barrier()
        pltpu.sync_copy(acc.at[pl.ds(row_lo, rows)],
                        out_hbm.at[pl.ds(cNp + row_lo, rows)])
        plsc.subcore_barrier()

    def run_pass(tbl, out_hbm, first_pass, ebody):
        zero_acc()

        def in_copies(w, b):
            base = tile_lo + w * K
            cps = [
                (src_hbm.at[pl.ds(base, K)], sbuf[b]),
                (dst_hbm.at[pl.ds(base, K)], dbuf[b]),
                (fb_hbm.at[pl.ds((coff_e + base) * 64, K * 64)], fbbuf[b]),
            ]
            if first_pass:
                cps.append((fa_hbm.at[pl.ds((coff_e + base) * 32, K * 32)], fabuf[b]))
                cps.append((u0_hbm.at[pl.ds(base, K)], ubuf[b].at[pl.ds(0, K)]))
            else:
                cps.append((u12_hbm.at[pl.ds(2 * base, 2 * K)], ubuf[b]))
            return cps

        def fire_in(w, b):
            for s_, d_ in in_copies(w, b):
                pltpu.async_copy(s_, d_, semi[b])

        def drain_in(w, b):
            for s_, d_ in in_copies(w, b):
                pltpu.make_async_copy(s_, d_, semi[b]).wait()

        def prep_gather(b):
            for i in range(K // 16):
                sl = pl.ds(i * 16, 16)
                gidx[b][sl] = sbuf[b][sl] + cN_vec
            pltpu.async_copy(tbl.at[gidx[b]], gbuf[b], semg[b])

        def drain_gather(b):
            pltpu.make_async_copy(tbl.at[gidx[b]], gbuf[b], semg[b]).wait()

        def fire_scatter(b):
            for i in range(K // 16):
                sl = pl.ds(i * 16, 16)
                sidx[b][sl] = dbuf[b][sl]
            pltpu.async_copy(obuf[b], acc.at[sidx[b]], sems[b], add=True)

        def drain_scatter(b):
            pltpu.make_async_copy(obuf[b], acc.at[sidx[b]], sems[b]).wait()

        fire_in(0, 0)
        fire_in(1, 1)
        drain_in(0, 0)
        prep_gather(0)

        def wpbody(wp, carry):
            for half in range(2):
                w = wp * 2 + half
                b = half
                b1 = 1 - half

                @pl.when(w + 1 < nwin)
                def _():
                    drain_in(w + 1, b1)
                    prep_gather(b1)

                drain_gather(b)

                @pl.when(w >= 2)
                def _():
                    drain_scatter(b)

                lax.fori_loop(0, K, ebody(b), 0)
                fire_scatter(b)

                @pl.when(w + 2 < nwin)
                def _():
                    fire_in(w + 2, b)
            return carry

        lax.fori_loop(0, nwin // 2, wpbody, 0)
        drain_scatter(0)
        drain_scatter(1)
        dump_acc(out_hbm)

    def _up(x):
        return plsc.unpack(x, format=plsc.PackFormat.INTERLEAVED)

    def _bc(x):
        return plsc.bitcast(x, jnp.bfloat16)

    def ebody1(b):
        def body(k, carry):
            u0 = plsc.load_gather(ubuf[b], [jnp.full((16,), 0, jnp.int32) + k])
            for j in range(2):
                s32 = pl.ds(j * 32, 32)
                lo = pl.ds(j * 32, 16)
                hi = pl.ds(j * 32 + 16, 16)
                tds0, tds1 = _up(_bc(gbuf[b][k, pl.ds(j * 16, 16)]))
                g00, g01 = _up(_bc(gbuf[b][k, pl.ds(32 + j * 16, 16)]))
                tvr0, tvr1 = _up(_bc(gbuf[b][k, pl.ds(64 + j * 16, 16)]))
                fds0, fds1 = _up(_bc(fabuf[b][pl.ds(k * 32 + j * 16, 16)]))
                fvv0, fvv1 = _up(_bc(fbbuf[b][pl.ds(k * 64 + j * 16, 16)]))
                fvr0, fvr1 = _up(_bc(fbbuf[b][pl.ds(k * 64 + 32 + j * 16, 16)]))
                obuf[b][k, lo] = fds0 * tds0
                obuf[b][k, hi] = fds1 * tds1
                obuf[b][k, pl.ds(64 + j * 32, 16)] = fvv0 * g00 + fvr0 * tvr0 * u0
                obuf[b][k, pl.ds(64 + j * 32 + 16, 16)] = fvv1 * g01 + fvr1 * tvr1 * u0
            return carry
        return body

    run_pass(t1_hbm, out1_hbm, True, ebody1)

    def ebody2(b):
        def body(k, carry):
            k2 = 2 * k
            u1 = plsc.load_gather(ubuf[b], [jnp.full((16,), 0, jnp.int32) + k2])
            u2 = plsc.load_gather(ubuf[b], [jnp.full((16,), 1, jnp.int32) + k2])
            for j in range(2):
                s32 = pl.ds(j * 32, 32)
                g10, g11 = _up(_bc(gbuf[b][k, pl.ds(j * 16, 16)]))
                g20, g21 = _up(_bc(gbuf[b][k, pl.ds(32 + j * 16, 16)]))
                tvr0, tvr1 = _up(_bc(gbuf[b][k, pl.ds(64 + j * 16, 16)]))
                fvv0, fvv1 = _up(_bc(fbbuf[b][pl.ds(k * 64 + j * 16, 16)]))
                fvr0, fvr1 = _up(_bc(fbbuf[b][pl.ds(k * 64 + 32 + j * 16, 16)]))
                mvr0 = fvr0 * tvr0
                mvr1 = fvr1 * tvr1
                obuf[b][k, pl.ds(j * 32, 16)] = fvv0 * g10 + mvr0 * u1
                obuf[b][k, pl.ds(j * 32 + 16, 16)] = fvv1 * g11 + mvr1 * u1
                obuf[b][k, pl.ds(64 + j * 32, 16)] = fvv0 * g20 + mvr0 * u2
                obuf[b][k, pl.ds(64 + j * 32 + 16, 16)] = fvv1 * g21 + mvr1 * u2
            return carry
        return body

    run_pass(t2_hbm, out2_hbm, False, ebody2)


def _sc_scatter(fa, fb, t1, t2, src, dst, u0, u12, zeros, N, Np, E):
    mesh = plsc.VectorSubcoreMesh(core_axis_name="c", subcore_axis_name="s")
    K = _K
    dbl = lambda mk: [mk(), mk()]
    kfn = functools.partial(
        pl.kernel,
        out_type=[
            jax.ShapeDtypeStruct((2 * Np, 128), jnp.float32),
            jax.ShapeDtypeStruct((2 * Np, 128), jnp.float32),
        ],
        mesh=mesh,
        scratch_types=(
            [pltpu.VMEM_SHARED((Np, 128), jnp.float32)]
            + dbl(lambda: pltpu.VMEM((K,), jnp.int32))
            + dbl(lambda: pltpu.VMEM((K,), jnp.int32))
            + dbl(lambda: pltpu.VMEM((K,), jnp.int32))
            + dbl(lambda: pltpu.VMEM((K,), jnp.int32))
            + dbl(lambda: pltpu.VMEM((2 * K,), jnp.float32))
            + dbl(lambda: pltpu.VMEM((K * 32,), jnp.int32))
            + dbl(lambda: pltpu.VMEM((K * 64,), jnp.int32))
            + dbl(lambda: pltpu.VMEM((K, 128), jnp.int32))
            + dbl(lambda: pltpu.VMEM((K, 128), jnp.float32))
            + [pltpu.SemaphoreType.DMA] * 6
        ),
        compiler_params=pltpu.CompilerParams(needs_layout_passes=False),
    )(functools.partial(_sc_body, N, Np, E))
    return kfn(fa, fb, t1, t2, src, dst, u0, u12, zeros)


def _update_body(s_ref, v_ref, o1_ref, o2_ref, uw_ref, vw_ref,
                 wu1_ref, bu1_ref, wu2_ref, bu2_ref, s_out_ref, v_out_ref):
    bN, _, H = v_ref.shape
    ds = jnp.concatenate([o1_ref[0][:, 0:64], o1_ref[1][:, 0:64]], axis=-1)
    dv0 = jnp.concatenate([o1_ref[0][:, 64:128], o1_ref[1][:, 64:128]], axis=-1)
    dv1 = jnp.concatenate([o2_ref[0][:, 0:64], o2_ref[1][:, 0:64]], axis=-1)
    dv2 = jnp.concatenate([o2_ref[0][:, 64:128], o2_ref[1][:, 64:128]], axis=-1)
    dv = jnp.concatenate([dv0[:, None, :], dv1[:, None, :], dv2[:, None, :]],
                         axis=1)
    s1 = s_ref[...] + ds
    v1 = v_ref[...] + dv
    v1f = v1.reshape(bN * 3, H)
    v_u = jnp.dot(v1f, uw_ref[...], preferred_element_type=jnp.float32)
    v_v = jnp.dot(v1f, vw_ref[...], preferred_element_type=jnp.float32)
    v_u = v_u.reshape(bN, 3, H)
    v_v = v_v.reshape(bN, 3, H)
    v_norm = jnp.sqrt(jnp.sum(v_v * v_v, axis=1))
    upd_in = jnp.concatenate([s1, v_norm], axis=-1)
    h = _silu(jnp.dot(upd_in, wu1_ref[...],
                      preferred_element_type=jnp.float32) + bu1_ref[...])
    out = jnp.dot(h, wu2_ref[...], preferred_element_type=jnp.float32) + bu2_ref[...]
    a = out[:, :H]
    b = out[:, H:2 * H]
    cc = out[:, 2 * H:]
    inner = jnp.sum(v_u * v_v, axis=1)
    s_out_ref[...] = s1 + a + b * inner
    v_out_ref[...] = v1 + cc[:, None, :] * v_u


def _update_phase(s, v, o1, o2, U_w, V_w, W_u1, b_u1, W_u2, b_u2, bN=400):
    N, H = s.shape
    grid = (N // bN,)
    return pl.pallas_call(
        _update_body,
        grid=grid,
        in_specs=[
            pl.BlockSpec((bN, H), lambda i: (i, 0)),
            pl.BlockSpec((bN, 3, H), lambda i: (i, 0, 0)),
            pl.BlockSpec((2, bN, 128), lambda i: (0, i, 0)),
            pl.BlockSpec((2, bN, 128), lambda i: (0, i, 0)),
            pl.BlockSpec(U_w.shape, lambda i: (0, 0)),
            pl.BlockSpec(V_w.shape, lambda i: (0, 0)),
            pl.BlockSpec(W_u1.shape, lambda i: (0, 0)),
            pl.BlockSpec(b_u1.shape, lambda i: (0,)),
            pl.BlockSpec(W_u2.shape, lambda i: (0, 0)),
            pl.BlockSpec(b_u2.shape, lambda i: (0,)),
        ],
        out_specs=[
            pl.BlockSpec((bN, H), lambda i: (i, 0)),
            pl.BlockSpec((bN, 3, H), lambda i: (i, 0, 0)),
        ],
        out_shape=[
            jax.ShapeDtypeStruct((N, H), jnp.float32),
            jax.ShapeDtypeStruct((N, 3, H), jnp.float32),
        ],
    )(s, v, o1, o2, U_w, V_w, W_u1, b_u1, W_u2, b_u2)


def kernel(s, v, edge_index, rbf, unit,
           W_f1, b_f1, W_f2, b_f2,
           W_s1, b_s1, W_s2, b_s2,
           U_w, V_w, W_u1, b_u1, W_u2, b_u2):
    N, H = s.shape
    E = edge_index.shape[1]
    src = edge_index[0]
    dst = edge_index[1]

    q32 = _np.empty(32, _np.int32)
    q32[0::2] = _np.arange(16)
    q32[1::2] = 16 + _np.arange(16)
    q64 = _np.concatenate([q32, 32 + q32])
    hperm = _np.concatenate([q64, 64 + q64])

    perm_outer = _np.concatenate([
        _np.arange(0, 128), _np.arange(128, 192), _np.arange(256, 320),
        _np.arange(192, 256), _np.arange(320, 384)])
    perm = _np.concatenate([perm_outer.reshape(6, 64)[i][q64] for i in range(6)])
    W_f2p = W_f2[:, perm]
    b_f2p = b_f2[perm]

    sperm = _np.concatenate([hperm, 128 + hperm, 256 + hperm])
    W_s2p = W_s2[:, sperm]
    b_s2p = b_s2[sperm]
    v_il = v[:, :, hperm]

    Np = 10112
    Ep = 327680
    npad = Ep - E
    rbf_p = jnp.pad(rbf, ((0, npad), (0, 0)))
    src_p = jnp.concatenate([src, jnp.arange(npad, dtype=jnp.int32) % N])
    dst_p = jnp.concatenate(
        [dst, N + (jnp.arange(npad, dtype=jnp.int32) % (Np - N))])
    unit_p = jnp.pad(unit, ((0, npad), (0, 0)))
    u0 = unit_p[:, 0]
    u12 = unit_p[:, 1:3].reshape(2 * Ep)

    fa, fb = _filter_mlp(rbf_p, W_f1, b_f1, W_f2p, b_f2p)
    t1, t2 = _node_tables(s, v_il, W_s1, b_s1, W_s2p, b_s2p)

    fa = lax.bitcast_convert_type(
        fa.reshape(2 * Ep * 32, 2), jnp.int32)
    fb = lax.bitcast_convert_type(
        fb.reshape(2 * Ep * 64, 2), jnp.int32)
    t1 = lax.bitcast_convert_type(
        t1.reshape(2 * N, 128, 2), jnp.int32)
    t2 = lax.bitcast_convert_type(
        t2.reshape(2 * N, 128, 2), jnp.int32)
    zeros = jnp.zeros((Np, 128), jnp.float32)

    o1, o2 = _sc_scatter(fa, fb, t1, t2, src_p, dst_p, u0, u12, zeros, N, Np, Ep)
    o1 = o1.reshape(2, Np, 128)
    o2 = o2.reshape(2, Np, 128)

    return _update_phase(s, v, o1, o2, U_w, V_w, W_u1, b_u1, W_u2, b_u2)


# speedup vs baseline: 14.7815x; 14.7815x over previous
import functools

import numpy as _np

import jax
import jax.numpy as jnp
from jax import lax
from jax.experimental import pallas as pl
from jax.experimental.pallas import tpu as pltpu
from jax.experimental.pallas import tpu_sc as plsc


def _silu(x):
    return x * jax.nn.sigmoid(x)


def _pack16(lo, hi):
    ul = lax.bitcast_convert_type(lo, jnp.uint32)
    uh = lax.bitcast_convert_type(hi, jnp.uint32)
    rl = (ul + jnp.uint32(0x7FFF) + ((ul >> 16) & jnp.uint32(1))) >> 16
    rh = (uh + jnp.uint32(0x7FFF) + ((uh >> 16) & jnp.uint32(1))) >> 16
    return lax.bitcast_convert_type(rl | (rh << 16), jnp.int32)


def _packcols(x):
    return jnp.concatenate(
        [_pack16(x[:, 32 * m:32 * m + 16], x[:, 32 * m + 16:32 * m + 32])
         for m in range(x.shape[1] // 32)], axis=-1)


def _filter_body(rbf_ref, w1_ref, b1_ref, w2_ref, b2_ref, fa_ref, fb_ref):
    h = _silu(jnp.dot(rbf_ref[...], w1_ref[...],
                      preferred_element_type=jnp.float32) + b1_ref[...])
    f = jnp.dot(h, w2_ref[...], preferred_element_type=jnp.float32) + b2_ref[...]
    fa_ref[0] = _packcols(f[:, 0:64])
    fa_ref[1] = _packcols(f[:, 64:128])
    fb_ref[0] = _packcols(f[:, 128:256])
    fb_ref[1] = _packcols(f[:, 256:384])


def _filter_mlp(rbf, W_f1, b_f1, W_f2p, b_f2p, bE=2048):
    E, R = rbf.shape
    grid = (E // bE,)
    return pl.pallas_call(
        _filter_body,
        grid=grid,
        in_specs=[
            pl.BlockSpec((bE, R), lambda i: (i, 0)),
            pl.BlockSpec(W_f1.shape, lambda i: (0, 0)),
            pl.BlockSpec(b_f1.shape, lambda i: (0,)),
            pl.BlockSpec(W_f2p.shape, lambda i: (0, 0)),
            pl.BlockSpec(b_f2p.shape, lambda i: (0,)),
        ],
        out_specs=[
            pl.BlockSpec((2, bE, 32), lambda i: (0, i, 0)),
            pl.BlockSpec((2, bE, 64), lambda i: (0, i, 0)),
        ],
        out_shape=[
            jax.ShapeDtypeStruct((2, E, 32), jnp.int32),
            jax.ShapeDtypeStruct((2, E, 64), jnp.int32),
        ],
    )(rbf, W_f1, b_f1, W_f2p, b_f2p)


def _tables_body(s_ref, v_ref, w1_ref, b1_ref, w2_ref, b2_ref, t1_ref, t2_ref):
    bN = s_ref.shape[0]
    h = _silu(jnp.dot(s_ref[...], w1_ref[...],
                      preferred_element_type=jnp.float32) + b1_ref[...])
    t = jnp.dot(h, w2_ref[...], preferred_element_type=jnp.float32) + b2_ref[...]
    v = v_ref[...]
    pad = jnp.zeros((bN, 64), jnp.float32)
    for c in range(2):
        tds = t[:, 64 * c:64 * c + 64]
        tvv = t[:, 128 + 64 * c:128 + 64 * c + 64]
        tvr = t[:, 256 + 64 * c:256 + 64 * c + 64]
        g0 = tvv * v[:, 0, 64 * c:64 * c + 64]
        g1 = tvv * v[:, 1, 64 * c:64 * c + 64]
        g2 = tvv * v[:, 2, 64 * c:64 * c + 64]
        t1_ref[c] = _packcols(jnp.concatenate([tds, g0, tvr, pad], axis=-1))
        t2_ref[c] = _packcols(jnp.concatenate([g1, g2, tvr, pad], axis=-1))


def _node_tables(s, v, W_s1, b_s1, W_s2, b_s2, bN=400):
    N, H = s.shape
    grid = (N // bN,)
    return pl.pallas_call(
        _tables_body,
        grid=grid,
        in_specs=[
            pl.BlockSpec((bN, H), lambda i: (i, 0)),
            pl.BlockSpec((bN, 3, H), lambda i: (i, 0, 0)),
            pl.BlockSpec(W_s1.shape, lambda i: (0, 0)),
            pl.BlockSpec(b_s1.shape, lambda i: (0,)),
            pl.BlockSpec(W_s2.shape, lambda i: (0, 0)),
            pl.BlockSpec(b_s2.shape, lambda i: (0,)),
        ],
        out_specs=[
            pl.BlockSpec((2, bN, 128), lambda i: (0, i, 0)),
            pl.BlockSpec((2, bN, 128), lambda i: (0, i, 0)),
        ],
        out_shape=[
            jax.ShapeDtypeStruct((2, N, 128), jnp.int32),
            jax.ShapeDtypeStruct((2, N, 128), jnp.int32),
        ],
    )(s, v, W_s1, b_s1, W_s2, b_s2)


_K = 64


def _sc_body(N, Np, E, fa_hbm, fb_hbm, t1_hbm, t2_hbm, src_hbm, dst_hbm,
             u0_hbm, u12_hbm, zeros_hbm, out1_hbm, out2_hbm,
             acc,
             sbuf0, sbuf1, dbuf0, dbuf1, gidx0, gidx1, sidx0, sidx1,
             ubuf0, ubuf1, fabuf0, fabuf1, fbbuf0, fbbuf1,
             gbuf0, gbuf1, obuf0, obuf1,
             semi0, semi1, semg0, semg1, sems0, sems1):
    c = lax.axis_index("c")
    sid = lax.axis_index("s")
    K = _K
    ept = E // 16
    nwin = ept // K
    tile_lo = sid * ept
    coff_e = c * E

    rows = Np // 16
    row_lo = sid * rows
    cNp = c * Np

    cN_vec = jnp.full((16,), c * N, jnp.int32)

    sbuf = [sbuf0, sbuf1]
    dbuf = [dbuf0, dbuf1]
    gidx = [gidx0, gidx1]
    sidx = [sidx0, sidx1]
    ubuf = [ubuf0, ubuf1]
    fabuf = [fabuf0, fabuf1]
    fbbuf = [fbbuf0, fbbuf1]
    gbuf = [gbuf0, gbuf1]
    obuf = [obuf0, obuf1]
    semi = [semi0, semi1]
    semg = [semg0, semg1]
    sems = [sems0, sems1]

    def zero_acc():
        pltpu.sync_copy(zeros_hbm.at[pl.ds(row_lo, rows)],
                        acc.at[pl.ds(row_lo, rows)])
        plsc.subcore_barrier()

    def dump_acc(out_hbm):
        plsc.subcore_barrier()
        pltpu.sync_copy(acc.at[pl.ds(row_lo, rows)],
                        out_hbm.at[pl.ds(cNp + row_lo, rows)])
        plsc.subcore_barrier()

    def run_pass(tbl, out_hbm, first_pass, ebody):
        zero_acc()

        def in_copies(w, b):
            base = tile_lo + w * K
            cps = [
                (src_hbm.at[pl.ds(base, K)], sbuf[b]),
                (dst_hbm.at[pl.ds(base, K)], dbuf[b]),
                (fb_hbm.at[pl.ds((coff_e + base) * 64, K * 64)], fbbuf[b]),
            ]
            if first_pass:
                cps.append((fa_hbm.at[pl.ds((coff_e + base) * 32, K * 32)], fabuf[b]))
                cps.append((u0_hbm.at[pl.ds(base, K)], ubuf[b].at[pl.ds(0, K)]))
            else:
                cps.append((u12_hbm.at[pl.ds(2 * base, 2 * K)], ubuf[b]))
            return cps

        def fire_in(w, b):
            for s_, d_ in in_copies(w, b):
                pltpu.async_copy(s_, d_, semi[b])

        def drain_in(w, b):
            for s_, d_ in in_copies(w, b):
                pltpu.make_async_copy(s_, d_, semi[b]).wait()

        def prep_gather(b):
            for i in range(K // 16):
                sl = pl.ds(i * 16, 16)
                gidx[b][sl] = sbuf[b][sl] + cN_vec
            pltpu.async_copy(tbl.at[gidx[b]], gbuf[b], semg[b])

        def drain_gather(b):
            pltpu.make_async_copy(tbl.at[gidx[b]], gbuf[b], semg[b]).wait()

        def fire_scatter(b):
            for i in range(K // 16):
                sl = pl.ds(i * 16, 16)
                sidx[b][sl] = dbuf[b][sl]
            pltpu.async_copy(obuf[b], acc.at[sidx[b]], sems[b], add=True)

        def drain_scatter(b):
            pltpu.make_async_copy(obuf[b], acc.at[sidx[b]], sems[b]).wait()

        fire_in(0, 0)
        fire_in(1, 1)
        drain_in(0, 0)
        prep_gather(0)

        def wpbody(wp, carry):
            for half in range(2):
                w = wp * 2 + half
                b = half
                b1 = 1 - half

                @pl.when(w + 1 < nwin)
                def _():
                    drain_in(w + 1, b1)
                    prep_gather(b1)

                drain_gather(b)

                @pl.when(w >= 2)
                def _():
                    drain_scatter(b)

                lax.fori_loop(0, K, ebody(b), 0)
                fire_scatter(b)

                @pl.when(w + 2 < nwin)
                def _():
                    fire_in(w + 2, b)
            return carry

        lax.fori_loop(0, nwin // 2, wpbody, 0)
        drain_scatter(0)
        drain_scatter(1)
        dump_acc(out_hbm)

    def _up(x):
        return plsc.unpack(x, format=plsc.PackFormat.INTERLEAVED)

    def _bc(x):
        return plsc.bitcast(x, jnp.bfloat16)

    def ebody1(b):
        def body(k, carry):
            u0 = plsc.load_gather(ubuf[b], [jnp.full((16,), 0, jnp.int32) + k])
            for j in range(2):
                s32 = pl.ds(j * 32, 32)
                lo = pl.ds(j * 32, 16)
                hi = pl.ds(j * 32 + 16, 16)
                tds0, tds1 = _up(_bc(gbuf[b][k, pl.ds(j * 16, 16)]))
                g00, g01 = _up(_bc(gbuf[b][k, pl.ds(32 + j * 16, 16)]))
                tvr0, tvr1 = _up(_bc(gbuf[b][k, pl.ds(64 + j * 16, 16)]))
                fds0, fds1 = _up(_bc(fabuf[b][pl.ds(k * 32 + j * 16, 16)]))
                fvv0, fvv1 = _up(_bc(fbbuf[b][pl.ds(k * 64 + j * 16, 16)]))
                fvr0, fvr1 = _up(_bc(fbbuf[b][pl.ds(k * 64 + 32 + j * 16, 16)]))
                obuf[b][k, lo] = fds0 * tds0
                obuf[b][k, hi] = fds1 * tds1
                obuf[b][k, pl.ds(64 + j * 32, 16)] = fvv0 * g00 + fvr0 * tvr0 * u0
                obuf[b][k, pl.ds(64 + j * 32 + 16, 16)] = fvv1 * g01 + fvr1 * tvr1 * u0
            return carry
        return body

    run_pass(t1_hbm, out1_hbm, True, ebody1)

    def ebody2(b):
        def body(k, carry):
            k2 = 2 * k
            u1 = plsc.load_gather(ubuf[b], [jnp.full((16,), 0, jnp.int32) + k2])
            u2 = plsc.load_gather(ubuf[b], [jnp.full((16,), 1, jnp.int32) + k2])
            for j in range(2):
                s32 = pl.ds(j * 32, 32)
                g10, g11 = _up(_bc(gbuf[b][k, pl.ds(j * 16, 16)]))
                g20, g21 = _up(_bc(gbuf[b][k, pl.ds(32 + j * 16, 16)]))
                tvr0, tvr1 = _up(_bc(gbuf[b][k, pl.ds(64 + j * 16, 16)]))
                fvv0, fvv1 = _up(_bc(fbbuf[b][pl.ds(k * 64 + j * 16, 16)]))
                fvr0, fvr1 = _up(_bc(fbbuf[b][pl.ds(k * 64 + 32 + j * 16, 16)]))
                mvr0 = fvr0 * tvr0
                mvr1 = fvr1 * tvr1
                obuf[b][k, pl.ds(j * 32, 16)] = fvv0 * g10 + mvr0 * u1
                obuf[b][k, pl.ds(j * 32 + 16, 16)] = fvv1 * g11 + mvr1 * u1
                obuf[b][k, pl.ds(64 + j * 32, 16)] = fvv0 * g20 + mvr0 * u2
                obuf[b][k, pl.ds(64 + j * 32 + 16, 16)] = fvv1 * g21 + mvr1 * u2
            return carry
        return body

    run_pass(t2_hbm, out2_hbm, False, ebody2)


def _sc_scatter(fa, fb, t1, t2, src, dst, u0, u12, zeros, N, Np, E):
    mesh = plsc.VectorSubcoreMesh(core_axis_name="c", subcore_axis_name="s")
    K = _K
    dbl = lambda mk: [mk(), mk()]
    kfn = functools.partial(
        pl.kernel,
        out_type=[
            jax.ShapeDtypeStruct((2 * Np, 128), jnp.float32),
            jax.ShapeDtypeStruct((2 * Np, 128), jnp.float32),
        ],
        mesh=mesh,
        scratch_types=(
            [pltpu.VMEM_SHARED((Np, 128), jnp.float32)]
            + dbl(lambda: pltpu.VMEM((K,), jnp.int32))
            + dbl(lambda: pltpu.VMEM((K,), jnp.int32))
            + dbl(lambda: pltpu.VMEM((K,), jnp.int32))
            + dbl(lambda: pltpu.VMEM((K,), jnp.int32))
            + dbl(lambda: pltpu.VMEM((2 * K,), jnp.float32))
            + dbl(lambda: pltpu.VMEM((K * 32,), jnp.int32))
            + dbl(lambda: pltpu.VMEM((K * 64,), jnp.int32))
            + dbl(lambda: pltpu.VMEM((K, 128), jnp.int32))
            + dbl(lambda: pltpu.VMEM((K, 128), jnp.float32))
            + [pltpu.SemaphoreType.DMA] * 6
        ),
        compiler_params=pltpu.CompilerParams(needs_layout_passes=False),
    )(functools.partial(_sc_body, N, Np, E))
    return kfn(fa, fb, t1, t2, src, dst, u0, u12, zeros)


def _update_body(s_ref, v_ref, o1_ref, o2_ref, uw_ref, vw_ref,
                 wu1_ref, bu1_ref, wu2_ref, bu2_ref, s_out_ref, v_out_ref):
    bN, _, H = v_ref.shape
    ds = jnp.concatenate([o1_ref[0][:, 0:64], o1_ref[1][:, 0:64]], axis=-1)
    dv0 = jnp.concatenate([o1_ref[0][:, 64:128], o1_ref[1][:, 64:128]], axis=-1)
    dv1 = jnp.concatenate([o2_ref[0][:, 0:64], o2_ref[1][:, 0:64]], axis=-1)
    dv2 = jnp.concatenate([o2_ref[0][:, 64:128], o2_ref[1][:, 64:128]], axis=-1)
    dv = jnp.concatenate([dv0[:, None, :], dv1[:, None, :], dv2[:, None, :]],
                         axis=1)
    s1 = s_ref[...] + ds
    v1 = v_ref[...] + dv
    v1f = v1.reshape(bN * 3, H)
    v_u = jnp.dot(v1f, uw_ref[...], preferred_element_type=jnp.float32)
    v_v = jnp.dot(v1f, vw_ref[...], preferred_element_type=jnp.float32)
    v_u = v_u.reshape(bN, 3, H)
    v_v = v_v.reshape(bN, 3, H)
    v_norm = jnp.sqrt(jnp.sum(v_v * v_v, axis=1))
    upd_in = jnp.concatenate([s1, v_norm], axis=-1)
    h = _silu(jnp.dot(upd_in, wu1_ref[...],
                      preferred_element_type=jnp.float32) + bu1_ref[...])
    out = jnp.dot(h, wu2_ref[...], preferred_element_type=jnp.float32) + bu2_ref[...]
    a = out[:, :H]
    b = out[:, H:2 * H]
    cc = out[:, 2 * H:]
    inner = jnp.sum(v_u * v_v, axis=1)
    s_out_ref[...] = s1 + a + b * inner
    v_out_ref[...] = v1 + cc[:, None, :] * v_u


def _update_phase(s, v, o1, o2, U_w, V_w, W_u1, b_u1, W_u2, b_u2, bN=400):
    N, H = s.shape
    grid = (N // bN,)
    return pl.pallas_call(
        _update_body,
        grid=grid,
        in_specs=[
            pl.BlockSpec((bN, H), lambda i: (i, 0)),
            pl.BlockSpec((bN, 3, H), lambda i: (i, 0, 0)),
            pl.BlockSpec((2, bN, 128), lambda i: (0, i, 0)),
            pl.BlockSpec((2, bN, 128), lambda i: (0, i, 0)),
            pl.BlockSpec(U_w.shape, lambda i: (0, 0)),
            pl.BlockSpec(V_w.shape, lambda i: (0, 0)),
            pl.BlockSpec(W_u1.shape, lambda i: (0, 0)),
            pl.BlockSpec(b_u1.shape, lambda i: (0,)),
            pl.BlockSpec(W_u2.shape, lambda i: (0, 0)),
            pl.BlockSpec(b_u2.shape, lambda i: (0,)),
        ],
        out_specs=[
            pl.BlockSpec((bN, H), lambda i: (i, 0)),
            pl.BlockSpec((bN, 3, H), lambda i: (i, 0, 0)),
        ],
        out_shape=[
            jax.ShapeDtypeStruct((N, H), jnp.float32),
            jax.ShapeDtypeStruct((N, 3, H), jnp.float32),
        ],
    )(s, v, o1, o2, U_w, V_w, W_u1, b_u1, W_u2, b_u2)


def kernel(s, v, edge_index, rbf, unit,
           W_f1, b_f1, W_f2, b_f2,
           W_s1, b_s1, W_s2, b_s2,
           U_w, V_w, W_u1, b_u1, W_u2, b_u2):
    N, H = s.shape
    E = edge_index.shape[1]
    src = edge_index[0]
    dst = edge_index[1]

    perm = _np.concatenate([
        _np.arange(0, 128), _np.arange(128, 192), _np.arange(256, 320),
        _np.arange(192, 256), _np.arange(320, 384)])
    W_f2p = W_f2[:, perm]
    b_f2p = b_f2[perm]

    Np = 10112
    Ep = 327680
    npad = Ep - E
    rbf_p = jnp.pad(rbf, ((0, npad), (0, 0)))
    src_p = jnp.concatenate([src, jnp.arange(npad, dtype=jnp.int32) % N])
    dst_p = jnp.concatenate(
        [dst, N + (jnp.arange(npad, dtype=jnp.int32) % (Np - N))])
    unit_p = jnp.pad(unit, ((0, npad), (0, 0)))
    u0 = unit_p[:, 0]
    u12 = unit_p[:, 1:3].reshape(2 * Ep)

    fa, fb = _filter_mlp(rbf_p, W_f1, b_f1, W_f2p, b_f2p)
    t1, t2 = _node_tables(s, v, W_s1, b_s1, W_s2, b_s2)

    fa = fa.reshape(2 * Ep * 32)
    fb = fb.reshape(2 * Ep * 64)
    t1 = t1.reshape(2 * N, 128)
    t2 = t2.reshape(2 * N, 128)
    zeros = jnp.zeros((Np, 128), jnp.float32)

    o1, o2 = _sc_scatter(fa, fb, t1, t2, src_p, dst_p, u0, u12, zeros, N, Np, Ep)
    o1 = o1.reshape(2, Np, 128)
    o2 = o2.reshape(2, Np, 128)

    return _update_phase(s, v, o1, o2, U_w, V_w, W_u1, b_u1, W_u2, b_u2)


# speedup vs baseline: 14.8931x; 1.0075x over previous
import functools

import numpy as _np

import jax
import jax.numpy as jnp
from jax import lax
from jax.experimental import pallas as pl
from jax.experimental.pallas import tpu as pltpu
from jax.experimental.pallas import tpu_sc as plsc


def _silu(x):
    return x * jax.nn.sigmoid(x)


def _pack16(lo, hi):
    ul = lax.bitcast_convert_type(lo, jnp.uint32)
    uh = lax.bitcast_convert_type(hi, jnp.uint32)
    rl = (ul + jnp.uint32(0x7FFF) + ((ul >> 16) & jnp.uint32(1))) >> 16
    rh = (uh + jnp.uint32(0x7FFF) + ((uh >> 16) & jnp.uint32(1))) >> 16
    return lax.bitcast_convert_type(rl | (rh << 16), jnp.int32)


def _packcols(x):
    return jnp.concatenate(
        [_pack16(x[:, 32 * m:32 * m + 16], x[:, 32 * m + 16:32 * m + 32])
         for m in range(x.shape[1] // 32)], axis=-1)


def _filter_body(rbf_ref, w1_ref, b1_ref, w2_ref, b2_ref, fa_ref, fb_ref):
    h = _silu(jnp.dot(rbf_ref[...], w1_ref[...],
                      preferred_element_type=jnp.float32) + b1_ref[...])
    f = jnp.dot(h, w2_ref[...], preferred_element_type=jnp.float32) + b2_ref[...]
    fa_ref[0] = _packcols(f[:, 0:64])
    fa_ref[1] = _packcols(f[:, 64:128])
    fb_ref[0] = _packcols(f[:, 128:256])
    fb_ref[1] = _packcols(f[:, 256:384])


def _filter_mlp(rbf, W_f1, b_f1, W_f2p, b_f2p, bE=5120):
    E, R = rbf.shape
    grid = (E // bE,)
    return pl.pallas_call(
        _filter_body,
        grid=grid,
        in_specs=[
            pl.BlockSpec((bE, R), lambda i: (i, 0)),
            pl.BlockSpec(W_f1.shape, lambda i: (0, 0)),
            pl.BlockSpec(b_f1.shape, lambda i: (0,)),
            pl.BlockSpec(W_f2p.shape, lambda i: (0, 0)),
            pl.BlockSpec(b_f2p.shape, lambda i: (0,)),
        ],
        out_specs=[
            pl.BlockSpec((2, bE, 32), lambda i: (0, i, 0)),
            pl.BlockSpec((2, bE, 64), lambda i: (0, i, 0)),
        ],
        out_shape=[
            jax.ShapeDtypeStruct((2, E, 32), jnp.int32),
            jax.ShapeDtypeStruct((2, E, 64), jnp.int32),
        ],
    )(rbf, W_f1, b_f1, W_f2p, b_f2p)


def _tables_body(s_ref, v_ref, w1_ref, b1_ref, w2_ref, b2_ref, t1_ref, t2_ref):
    bN = s_ref.shape[0]
    h = _silu(jnp.dot(s_ref[...], w1_ref[...],
                      preferred_element_type=jnp.float32) + b1_ref[...])
    t = jnp.dot(h, w2_ref[...], preferred_element_type=jnp.float32) + b2_ref[...]
    v = v_ref[...]
    pad = jnp.zeros((bN, 64), jnp.float32)
    for c in range(2):
        tds = t[:, 64 * c:64 * c + 64]
        tvv = t[:, 128 + 64 * c:128 + 64 * c + 64]
        tvr = t[:, 256 + 64 * c:256 + 64 * c + 64]
        g0 = tvv * v[:, 0, 64 * c:64 * c + 64]
        g1 = tvv * v[:, 1, 64 * c:64 * c + 64]
        g2 = tvv * v[:, 2, 64 * c:64 * c + 64]
        t1_ref[c] = _packcols(jnp.concatenate([tds, g0, tvr, pad], axis=-1))
        t2_ref[c] = _packcols(jnp.concatenate([g1, g2, tvr, pad], axis=-1))


def _node_tables(s, v, W_s1, b_s1, W_s2, b_s2, bN=1000):
    N, H = s.shape
    grid = (N // bN,)
    return pl.pallas_call(
        _tables_body,
        grid=grid,
        in_specs=[
            pl.BlockSpec((bN, H), lambda i: (i, 0)),
            pl.BlockSpec((bN, 3, H), lambda i: (i, 0, 0)),
            pl.BlockSpec(W_s1.shape, lambda i: (0, 0)),
            pl.BlockSpec(b_s1.shape, lambda i: (0,)),
            pl.BlockSpec(W_s2.shape, lambda i: (0, 0)),
            pl.BlockSpec(b_s2.shape, lambda i: (0,)),
        ],
        out_specs=[
            pl.BlockSpec((2, bN, 128), lambda i: (0, i, 0)),
            pl.BlockSpec((2, bN, 128), lambda i: (0, i, 0)),
        ],
        out_shape=[
            jax.ShapeDtypeStruct((2, N, 128), jnp.int32),
            jax.ShapeDtypeStruct((2, N, 128), jnp.int32),
        ],
    )(s, v, W_s1, b_s1, W_s2, b_s2)


_K = 64


def _sc_body(N, Np, E, fa_hbm, fb_hbm, t1_hbm, t2_hbm, src_hbm, dst_hbm,
             u0_hbm, u12_hbm, zeros_hbm, out1_hbm, out2_hbm,
             acc,
             sbuf0, sbuf1, dbuf0, dbuf1, gidx0, gidx1, sidx0, sidx1,
             ubuf0, ubuf1, fabuf0, fabuf1, fbbuf0, fbbuf1,
             gbuf0, gbuf1, obuf0, obuf1,
             semi0, semi1, semg0, semg1, sems0, sems1):
    c = lax.axis_index("c")
    sid = lax.axis_index("s")
    K = _K
    ept = E // 16
    nwin = ept // K
    tile_lo = sid * ept
    coff_e = c * E

    rows = Np // 16
    row_lo = sid * rows
    cNp = c * Np

    cN_vec = jnp.full((16,), c * N, jnp.int32)

    sbuf = [sbuf0, sbuf1]
    dbuf = [dbuf0, dbuf1]
    gidx = [gidx0, gidx1]
    sidx = [sidx0, sidx1]
    ubuf = [ubuf0, ubuf1]
    fabuf = [fabuf0, fabuf1]
    fbbuf = [fbbuf0, fbbuf1]
    gbuf = [gbuf0, gbuf1]
    obuf = [obuf0, obuf1]
    semi = [semi0, semi1]
    semg = [semg0, semg1]
    sems = [sems0, sems1]

    def zero_acc():
        pltpu.sync_copy(zeros_hbm.at[pl.ds(row_lo, rows)],
                        acc.at[pl.ds(row_lo, rows)])
        plsc.subcore_barrier()

    def dump_acc(out_hbm):
        plsc.subcore_barrier()
        pltpu.sync_copy(acc.at[pl.ds(row_lo, rows)],
                        out_hbm.at[pl.ds(cNp + row_lo, rows)])
        plsc.subcore_barrier()

    def run_pass(tbl, out_hbm, first_pass, ebody):
        zero_acc()

        def in_copies(w, b):
            base = tile_lo + w * K
            cps = [
                (src_hbm.at[pl.ds(base, K)], sbuf[b]),
                (dst_hbm.at[pl.ds(base, K)], dbuf[b]),
                (fb_hbm.at[pl.ds((coff_e + base) * 64, K * 64)], fbbuf[b]),
            ]
            if first_pass:
                cps.append((fa_hbm.at[pl.ds((coff_e + base) * 32, K * 32)], fabuf[b]))
                cps.append((u0_hbm.at[pl.ds(base, K)], ubuf[b].at[pl.ds(0, K)]))
            else:
                cps.append((u12_hbm.at[pl.ds(2 * base, 2 * K)], ubuf[b]))
            return cps

        def fire_in(w, b):
            for s_, d_ in in_copies(w, b):
                pltpu.async_copy(s_, d_, semi[b])

        def drain_in(w, b):
            for s_, d_ in in_copies(w, b):
                pltpu.make_async_copy(s_, d_, semi[b]).wait()

        def prep_gather(b):
            for i in range(K // 16):
                sl = pl.ds(i * 16, 16)
                gidx[b][sl] = sbuf[b][sl] + cN_vec
            pltpu.async_copy(tbl.at[gidx[b]], gbuf[b], semg[b])

        def drain_gather(b):
            pltpu.make_async_copy(tbl.at[gidx[b]], gbuf[b], semg[b]).wait()

        def fire_scatter(b):
            for i in range(K // 16):
                sl = pl.ds(i * 16, 16)
                sidx[b][sl] = dbuf[b][sl]
            pltpu.async_copy(obuf[b], acc.at[sidx[b]], sems[b], add=True)

        def drain_scatter(b):
            pltpu.make_async_copy(obuf[b], acc.at[sidx[b]], sems[b]).wait()

        fire_in(0, 0)
        fire_in(1, 1)
        drain_in(0, 0)
        prep_gather(0)

        def wpbody(wp, carry):
            for half in range(2):
                w = wp * 2 + half
                b = half
                b1 = 1 - half

                @pl.when(w + 1 < nwin)
                def _():
                    drain_in(w + 1, b1)
                    prep_gather(b1)

                drain_gather(b)

                @pl.when(w >= 2)
                def _():
                    drain_scatter(b)

                lax.fori_loop(0, K, ebody(b), 0)
                fire_scatter(b)

                @pl.when(w + 2 < nwin)
                def _():
                    fire_in(w + 2, b)
            return carry

        lax.fori_loop(0, nwin // 2, wpbody, 0)
        drain_scatter(0)
        drain_scatter(1)
        dump_acc(out_hbm)

    def _up(x):
        return plsc.unpack(x, format=plsc.PackFormat.INTERLEAVED)

    def _bc(x):
        return plsc.bitcast(x, jnp.bfloat16)

    def ebody1(b):
        def body(k, carry):
            u0 = plsc.load_gather(ubuf[b], [jnp.full((16,), 0, jnp.int32) + k])
            for j in range(2):
                s32 = pl.ds(j * 32, 32)
                lo = pl.ds(j * 32, 16)
                hi = pl.ds(j * 32 + 16, 16)
                tds0, tds1 = _up(_bc(gbuf[b][k, pl.ds(j * 16, 16)]))
                g00, g01 = _up(_bc(gbuf[b][k, pl.ds(32 + j * 16, 16)]))
                tvr0, tvr1 = _up(_bc(gbuf[b][k, pl.ds(64 + j * 16, 16)]))
                fds0, fds1 = _up(_bc(fabuf[b][pl.ds(k * 32 + j * 16, 16)]))
                fvv0, fvv1 = _up(_bc(fbbuf[b][pl.ds(k * 64 + j * 16, 16)]))
                fvr0, fvr1 = _up(_bc(fbbuf[b][pl.ds(k * 64 + 32 + j * 16, 16)]))
                obuf[b][k, lo] = fds0 * tds0
                obuf[b][k, hi] = fds1 * tds1
                obuf[b][k, pl.ds(64 + j * 32, 16)] = fvv0 * g00 + fvr0 * tvr0 * u0
                obuf[b][k, pl.ds(64 + j * 32 + 16, 16)] = fvv1 * g01 + fvr1 * tvr1 * u0
            return carry
        return body

    run_pass(t1_hbm, out1_hbm, True, ebody1)

    def ebody2(b):
        def body(k, carry):
            k2 = 2 * k
            u1 = plsc.load_gather(ubuf[b], [jnp.full((16,), 0, jnp.int32) + k2])
            u2 = plsc.load_gather(ubuf[b], [jnp.full((16,), 1, jnp.int32) + k2])
            for j in range(2):
                s32 = pl.ds(j * 32, 32)
                g10, g11 = _up(_bc(gbuf[b][k, pl.ds(j * 16, 16)]))
                g20, g21 = _up(_bc(gbuf[b][k, pl.ds(32 + j * 16, 16)]))
                tvr0, tvr1 = _up(_bc(gbuf[b][k, pl.ds(64 + j * 16, 16)]))
                fvv0, fvv1 = _up(_bc(fbbuf[b][pl.ds(k * 64 + j * 16, 16)]))
                fvr0, fvr1 = _up(_bc(fbbuf[b][pl.ds(k * 64 + 32 + j * 16, 16)]))
                mvr0 = fvr0 * tvr0
                mvr1 = fvr1 * tvr1
                obuf[b][k, pl.ds(j * 32, 16)] = fvv0 * g10 + mvr0 * u1
                obuf[b][k, pl.ds(j * 32 + 16, 16)] = fvv1 * g11 + mvr1 * u1
                obuf[b][k, pl.ds(64 + j * 32, 16)] = fvv0 * g20 + mvr0 * u2
                obuf[b][k, pl.ds(64 + j * 32 + 16, 16)] = fvv1 * g21 + mvr1 * u2
            return carry
        return body

    run_pass(t2_hbm, out2_hbm, False, ebody2)


def _sc_scatter(fa, fb, t1, t2, src, dst, u0, u12, zeros, N, Np, E):
    mesh = plsc.VectorSubcoreMesh(core_axis_name="c", subcore_axis_name="s")
    K = _K
    dbl = lambda mk: [mk(), mk()]
    kfn = functools.partial(
        pl.kernel,
        out_type=[
            jax.ShapeDtypeStruct((2 * Np, 128), jnp.float32),
            jax.ShapeDtypeStruct((2 * Np, 128), jnp.float32),
        ],
        mesh=mesh,
        scratch_types=(
            [pltpu.VMEM_SHARED((Np, 128), jnp.float32)]
            + dbl(lambda: pltpu.VMEM((K,), jnp.int32))
            + dbl(lambda: pltpu.VMEM((K,), jnp.int32))
            + dbl(lambda: pltpu.VMEM((K,), jnp.int32))
            + dbl(lambda: pltpu.VMEM((K,), jnp.int32))
            + dbl(lambda: pltpu.VMEM((2 * K,), jnp.float32))
            + dbl(lambda: pltpu.VMEM((K * 32,), jnp.int32))
            + dbl(lambda: pltpu.VMEM((K * 64,), jnp.int32))
            + dbl(lambda: pltpu.VMEM((K, 128), jnp.int32))
            + dbl(lambda: pltpu.VMEM((K, 128), jnp.float32))
            + [pltpu.SemaphoreType.DMA] * 6
        ),
        compiler_params=pltpu.CompilerParams(needs_layout_passes=False),
    )(functools.partial(_sc_body, N, Np, E))
    return kfn(fa, fb, t1, t2, src, dst, u0, u12, zeros)


def _update_body(s_ref, v_ref, o1_ref, o2_ref, uw_ref, vw_ref,
                 wu1_ref, bu1_ref, wu2_ref, bu2_ref, s_out_ref, v_out_ref):
    bN, _, H = v_ref.shape
    ds = jnp.concatenate([o1_ref[0][:, 0:64], o1_ref[1][:, 0:64]], axis=-1)
    dv0 = jnp.concatenate([o1_ref[0][:, 64:128], o1_ref[1][:, 64:128]], axis=-1)
    dv1 = jnp.concatenate([o2_ref[0][:, 0:64], o2_ref[1][:, 0:64]], axis=-1)
    dv2 = jnp.concatenate([o2_ref[0][:, 64:128], o2_ref[1][:, 64:128]], axis=-1)
    dv = jnp.concatenate([dv0[:, None, :], dv1[:, None, :], dv2[:, None, :]],
                         axis=1)
    s1 = s_ref[...] + ds
    v1 = v_ref[...] + dv
    v1f = v1.reshape(bN * 3, H)
    v_u = jnp.dot(v1f, uw_ref[...], preferred_element_type=jnp.float32)
    v_v = jnp.dot(v1f, vw_ref[...], preferred_element_type=jnp.float32)
    v_u = v_u.reshape(bN, 3, H)
    v_v = v_v.reshape(bN, 3, H)
    v_norm = jnp.sqrt(jnp.sum(v_v * v_v, axis=1))
    upd_in = jnp.concatenate([s1, v_norm], axis=-1)
    h = _silu(jnp.dot(upd_in, wu1_ref[...],
                      preferred_element_type=jnp.float32) + bu1_ref[...])
    out = jnp.dot(h, wu2_ref[...], preferred_element_type=jnp.float32) + bu2_ref[...]
    a = out[:, :H]
    b = out[:, H:2 * H]
    cc = out[:, 2 * H:]
    inner = jnp.sum(v_u * v_v, axis=1)
    s_out_ref[...] = s1 + a + b * inner
    v_out_ref[...] = v1 + cc[:, None, :] * v_u


def _update_phase(s, v, o1, o2, U_w, V_w, W_u1, b_u1, W_u2, b_u2, bN=1000):
    N, H = s.shape
    grid = (N // bN,)
    return pl.pallas_call(
        _update_body,
        grid=grid,
        in_specs=[
            pl.BlockSpec((bN, H), lambda i: (i, 0)),
            pl.BlockSpec((bN, 3, H), lambda i: (i, 0, 0)),
            pl.BlockSpec((2, bN, 128), lambda i: (0, i, 0)),
            pl.BlockSpec((2, bN, 128), lambda i: (0, i, 0)),
            pl.BlockSpec(U_w.shape, lambda i: (0, 0)),
            pl.BlockSpec(V_w.shape, lambda i: (0, 0)),
            pl.BlockSpec(W_u1.shape, lambda i: (0, 0)),
            pl.BlockSpec(b_u1.shape, lambda i: (0,)),
            pl.BlockSpec(W_u2.shape, lambda i: (0, 0)),
            pl.BlockSpec(b_u2.shape, lambda i: (0,)),
        ],
        out_specs=[
            pl.BlockSpec((bN, H), lambda i: (i, 0)),
            pl.BlockSpec((bN, 3, H), lambda i: (i, 0, 0)),
        ],
        out_shape=[
            jax.ShapeDtypeStruct((N, H), jnp.float32),
            jax.ShapeDtypeStruct((N, 3, H), jnp.float32),
        ],
    )(s, v, o1, o2, U_w, V_w, W_u1, b_u1, W_u2, b_u2)


def kernel(s, v, edge_index, rbf, unit,
           W_f1, b_f1, W_f2, b_f2,
           W_s1, b_s1, W_s2, b_s2,
           U_w, V_w, W_u1, b_u1, W_u2, b_u2):
    N, H = s.shape
    E = edge_index.shape[1]
    src = edge_index[0]
    dst = edge_index[1]

    perm = _np.concatenate([
        _np.arange(0, 128), _np.arange(128, 192), _np.arange(256, 320),
        _np.arange(192, 256), _np.arange(320, 384)])
    W_f2p = W_f2[:, perm]
    b_f2p = b_f2[perm]

    Np = 10112
    Ep = 327680
    npad = Ep - E
    rbf_p = jnp.pad(rbf, ((0, npad), (0, 0)))
    src_p = jnp.concatenate([src, jnp.arange(npad, dtype=jnp.int32) % N])
    dst_p = jnp.concatenate(
        [dst, N + (jnp.arange(npad, dtype=jnp.int32) % (Np - N))])
    unit_p = jnp.pad(unit, ((0, npad), (0, 0)))
    u0 = unit_p[:, 0]
    u12 = unit_p[:, 1:3].reshape(2 * Ep)

    fa, fb = _filter_mlp(rbf_p, W_f1, b_f1, W_f2p, b_f2p)
    t1, t2 = _node_tables(s, v, W_s1, b_s1, W_s2, b_s2)

    fa = fa.reshape(2 * Ep * 32)
    fb = fb.reshape(2 * Ep * 64)
    t1 = t1.reshape(2 * N, 128)
    t2 = t2.reshape(2 * N, 128)
    zeros = jnp.zeros((Np, 128), jnp.float32)

    o1, o2 = _sc_scatter(fa, fb, t1, t2, src_p, dst_p, u0, u12, zeros, N, Np, Ep)
    o1 = o1.reshape(2, Np, 128)
    o2 = o2.reshape(2, Np, 128)

    return _update_phase(s, v, o1, o2, U_w, V_w, W_u1, b_u1, W_u2, b_u2)
